# baseline (speedup 1.0000x reference)
import jax
import jax.numpy as jnp
from jax import lax
from jax.experimental import pallas as pl
from jax.experimental.pallas import tpu as pltpu
from jax.experimental.pallas import tpu_sc as plsc


def _tc_body(coords_ref, pr_ref, spatial_ref, gia_ref, ev_ref, tv_ref, mm_ref):
    H = W = 512
    S = coords_ref.shape[2]
    k = pl.program_id(1)
    s = spatial_ref[0]
    pxmin = jnp.min(s[0])
    pxmax = jnp.max(s[0])
    pymin = jnp.min(s[1])
    pymax = jnp.max(s[1])

    @pl.when(k == 0)
    def _():
        mm_ref[0] = pxmin
        mm_ref[1] = pxmax
        mm_ref[2] = pymin
        mm_ref[3] = pymax

    @pl.when(k > 0)
    def _():
        mm_ref[0] = jnp.minimum(mm_ref[0], pxmin)
        mm_ref[1] = jnp.maximum(mm_ref[1], pxmax)
        mm_ref[2] = jnp.minimum(mm_ref[2], pymin)
        mm_ref[3] = jnp.maximum(mm_ref[3], pymax)

    @pl.when(k == 3)
    def _():
        xmin = mm_ref[0]
        xmax = mm_ref[1]
        ymin = mm_ref[2]
        ymax = mm_ref[3]
        c = coords_ref[0]
        cx = c[0:1, :]
        cy = c[1:2, :]
        nx = (cx - xmin) / (xmax - xmin)
        ny = (cy - ymin) / (ymax - ymin)
        fx = jnp.clip(jnp.round(nx * (W - 1)), 0.0, W - 1)
        fy = jnp.clip(jnp.round(ny * (H - 1)), 0.0, H - 1)
        px = fx.astype(jnp.int32)
        py = fy.astype(jnp.int32)
        in_view = ((cx >= xmin) & (cx <= xmax) & (cy >= ymin) & (cy <= ymax))
        ih = lax.broadcasted_iota(jnp.int32, (H, S), 0)
        iw = lax.broadcasted_iota(jnp.int32, (W, S), 0)
        oh_y = (ih == py).astype(jnp.float32)
        oh_x = (iw == px).astype(jnp.float32)
        gia = gia_ref[0]
        rows = lax.dot_general(gia, oh_y, (((0,), (0,)), ((), ())),
                               preferred_element_type=jnp.float32)
        g = jnp.sum(rows * oh_x, axis=0, keepdims=True)
        e = pr_ref[0] * g * in_view.astype(jnp.float32)
        flat = py * W + px
        ev_ref[0] = jnp.broadcast_to(e, (8, S))
        tv_ref[0] = jnp.broadcast_to(flat, (8, S))


_NW = 32
_NQ = 8
_QW = 512 * 512 // _NQ
_QR = 512 // _NQ


def _sc_body(ev_hbm, tv_hbm, out_hbm, acc0, acc1, evv, tvv, sem0, sem1):
    cid = lax.axis_index("c")
    sid = lax.axis_index("s")
    wid = sid * 2 + cid
    zeros16 = jnp.zeros((16,), jnp.float32)
    bufs = (acc0, acc1)
    sems = (sem0, sem1)
    copies = [None, None]
    for p in range(4):
        acc = bufs[p % 2]
        if p >= 2:
            copies[p % 2].wait()
        item = wid + _NW * p
        b = item // _NQ
        q = item % _NQ

        def _zero(i, carry):
            for k in range(2):
                for m in range(32):
                    acc[i * 2 + k, pl.ds(m * 16, 16)] = zeros16
            return carry
        lax.fori_loop(0, _QR // 2, _zero, 0)

        pltpu.sync_copy(ev_hbm.at[b], evv)
        pltpu.sync_copy(tv_hbm.at[b], tvv)
        qbase = q * _QW
        for i in range(16):
            csl = pl.ds(i * 16, 16)
            ei = evv[0, csl]
            ti = tvv[0, csl]
            local = ti - qbase
            mask = (ei != 0.0) & (local >= 0) & (local < _QW)
            lc = jnp.minimum(jnp.maximum(local, 0), _QW - 1)
            plsc.addupdate_scatter(acc, [lax.shift_right_logical(lc, 9),
                                         jnp.bitwise_and(lc, 511)],
                                   ei, mask=mask)
        copies[p % 2] = pltpu.async_copy(
            acc, out_hbm.at[b, pl.ds(q * _QR, _QR)], sems[p % 2])
    copies[0].wait()
    copies[1].wait()


def kernel(point_rates, spatial, gia, all_source_coords):
    B, H, W = gia.shape
    S = all_source_coords.shape[0]
    coords3 = jnp.transpose(all_source_coords)[None]
    pr3 = point_rates[:, None, :]
    ev, tv = pl.pallas_call(
        _tc_body,
        grid=(B, 4),
        in_specs=[
            pl.BlockSpec((1, 2, S), lambda b, k: (0, 0, 0)),
            pl.BlockSpec((1, 1, S), lambda b, k: (b, 0, 0)),
            pl.BlockSpec((1, 2, H // 4, W), lambda b, k: (b, 0, k, 0)),
            pl.BlockSpec((1, H, W), lambda b, k: (b, 0, 0)),
        ],
        out_specs=[
            pl.BlockSpec((1, 8, S), lambda b, k: (b, 0, 0)),
            pl.BlockSpec((1, 8, S), lambda b, k: (b, 0, 0)),
        ],
        out_shape=[
            jax.ShapeDtypeStruct((B, 8, S), jnp.float32),
            jax.ShapeDtypeStruct((B, 8, S), jnp.int32),
        ],
        scratch_shapes=[pltpu.SMEM((4,), jnp.float32)],
    )(coords3, pr3, spatial, gia)
    mesh = plsc.VectorSubcoreMesh(core_axis_name="c", subcore_axis_name="s")
    sc = pl.kernel(
        _sc_body,
        out_type=jax.ShapeDtypeStruct((B, H, W), jnp.float32),
        mesh=mesh,
        compiler_params=pltpu.CompilerParams(needs_layout_passes=False),
        scratch_types=[
            pltpu.VMEM((_QR, 512), jnp.float32),
            pltpu.VMEM((_QR, 512), jnp.float32),
            pltpu.VMEM((8, S), jnp.float32),
            pltpu.VMEM((8, S), jnp.int32),
            pltpu.SemaphoreType.DMA,
            pltpu.SemaphoreType.DMA,
        ],
    )
    out3 = sc(ev, tv)
    return out3[:, None]


# speedup vs baseline: 1.4519x; 1.4519x over previous
import jax
import jax.numpy as jnp
from jax import lax
from jax.experimental import pallas as pl
from jax.experimental.pallas import tpu as pltpu
from jax.experimental.pallas import tpu_sc as plsc


def _tc_body(coords_ref, pr_ref, spatial_ref, gia_ref, ev_ref, tv_ref):
    H = W = 512
    S = coords_ref.shape[2]
    s = spatial_ref[0]
    xmin = jnp.min(s[0])
    xmax = jnp.max(s[0])
    ymin = jnp.min(s[1])
    ymax = jnp.max(s[1])
    if True:
        c = coords_ref[0]
        cx = c[0:1, :]
        cy = c[1:2, :]
        nx = (cx - xmin) / (xmax - xmin)
        ny = (cy - ymin) / (ymax - ymin)
        fx = jnp.clip(jnp.round(nx * (W - 1)), 0.0, W - 1)
        fy = jnp.clip(jnp.round(ny * (H - 1)), 0.0, H - 1)
        px = fx.astype(jnp.int32)
        py = fy.astype(jnp.int32)
        in_view = ((cx >= xmin) & (cx <= xmax) & (cy >= ymin) & (cy <= ymax))
        ih = lax.broadcasted_iota(jnp.int32, (H, S), 0)
        iw = lax.broadcasted_iota(jnp.int32, (W, S), 0)
        oh_y = (ih == py).astype(jnp.float32)
        oh_x = (iw == px).astype(jnp.float32)
        gia = gia_ref[0]
        rows = lax.dot_general(gia, oh_y, (((0,), (0,)), ((), ())),
                               preferred_element_type=jnp.float32)
        g = jnp.sum(rows * oh_x, axis=0, keepdims=True)
        e = pr_ref[0] * g * in_view.astype(jnp.float32)
        flat = py * W + px
        ev_ref[0] = jnp.broadcast_to(e, (8, S))
        tv_ref[0] = jnp.broadcast_to(flat, (8, S))


_NW = 32
_NQ = 8
_QW = 512 * 512 // _NQ
_QR = 512 // _NQ


def _sc_body(ev_hbm, tv_hbm, out_hbm, acc0, acc1, evv, tvv, sem0, sem1):
    cid = lax.axis_index("c")
    sid = lax.axis_index("s")
    wid = sid * 2 + cid
    zeros16 = jnp.zeros((16,), jnp.float32)
    bufs = (acc0, acc1)
    sems = (sem0, sem1)
    copies = [None, None]
    for p in range(4):
        acc = bufs[p % 2]
        if p >= 2:
            copies[p % 2].wait()
        item = wid + _NW * p
        b = item // _NQ
        q = item % _NQ

        def _zero(i, carry):
            for k in range(2):
                for m in range(32):
                    acc[i * 2 + k, pl.ds(m * 16, 16)] = zeros16
            return carry
        lax.fori_loop(0, _QR // 2, _zero, 0)

        pltpu.sync_copy(ev_hbm.at[b], evv)
        pltpu.sync_copy(tv_hbm.at[b], tvv)
        qbase = q * _QW
        for i in range(16):
            csl = pl.ds(i * 16, 16)
            ei = evv[0, csl]
            ti = tvv[0, csl]
            local = ti - qbase
            mask = (ei != 0.0) & (local >= 0) & (local < _QW)
            lc = jnp.minimum(jnp.maximum(local, 0), _QW - 1)
            plsc.addupdate_scatter(acc, [lax.shift_right_logical(lc, 9),
                                         jnp.bitwise_and(lc, 511)],
                                   ei, mask=mask)
        copies[p % 2] = pltpu.async_copy(
            acc, out_hbm.at[b, pl.ds(q * _QR, _QR)], sems[p % 2])
    copies[0].wait()
    copies[1].wait()


def kernel(point_rates, spatial, gia, all_source_coords):
    B, H, W = gia.shape
    S = all_source_coords.shape[0]
    coords3 = jnp.transpose(all_source_coords)[None]
    pr3 = point_rates[:, None, :]
    ev, tv = pl.pallas_call(
        _tc_body,
        grid=(B,),
        in_specs=[
            pl.BlockSpec((1, 2, S), lambda b: (0, 0, 0)),
            pl.BlockSpec((1, 1, S), lambda b: (b, 0, 0)),
            pl.BlockSpec((1, 2, H, W), lambda b: (b, 0, 0, 0)),
            pl.BlockSpec((1, H, W), lambda b: (b, 0, 0)),
        ],
        out_specs=[
            pl.BlockSpec((1, 8, S), lambda b: (b, 0, 0)),
            pl.BlockSpec((1, 8, S), lambda b: (b, 0, 0)),
        ],
        out_shape=[
            jax.ShapeDtypeStruct((B, 8, S), jnp.float32),
            jax.ShapeDtypeStruct((B, 8, S), jnp.int32),
        ],
    )(coords3, pr3, spatial, gia)
    mesh = plsc.VectorSubcoreMesh(core_axis_name="c", subcore_axis_name="s")
    sc = pl.kernel(
        _sc_body,
        out_type=jax.ShapeDtypeStruct((B, H, W), jnp.float32),
        mesh=mesh,
        compiler_params=pltpu.CompilerParams(needs_layout_passes=False, skip_device_barrier=True),
        scratch_types=[
            pltpu.VMEM((_QR, 512), jnp.float32),
            pltpu.VMEM((_QR, 512), jnp.float32),
            pltpu.VMEM((8, S), jnp.float32),
            pltpu.VMEM((8, S), jnp.int32),
            pltpu.SemaphoreType.DMA,
            pltpu.SemaphoreType.DMA,
        ],
    )
    out3 = sc(ev, tv)
    return out3[:, None]


# speedup vs baseline: 1.6792x; 1.1566x over previous
import jax
import jax.numpy as jnp
from jax import lax
from jax.experimental import pallas as pl
from jax.experimental.pallas import tpu as pltpu
from jax.experimental.pallas import tpu_sc as plsc


def _tc_body(coords_ref, pr_ref, spatial_ref, gia_ref, ev_ref, tv_ref):
    H = W = 512
    S = coords_ref.shape[2]
    s = spatial_ref[0]
    xmin = jnp.min(s[0])
    xmax = jnp.max(s[0])
    ymin = jnp.min(s[1])
    ymax = jnp.max(s[1])
    if True:
        c = coords_ref[0]
        cx = c[0:1, :]
        cy = c[1:2, :]
        nx = (cx - xmin) / (xmax - xmin)
        ny = (cy - ymin) / (ymax - ymin)
        fx = jnp.clip(jnp.round(nx * (W - 1)), 0.0, W - 1)
        fy = jnp.clip(jnp.round(ny * (H - 1)), 0.0, H - 1)
        px = fx.astype(jnp.int32)
        py = fy.astype(jnp.int32)
        in_view = ((cx >= xmin) & (cx <= xmax) & (cy >= ymin) & (cy <= ymax))
        ih = lax.broadcasted_iota(jnp.int32, (H, S), 0)
        iw = lax.broadcasted_iota(jnp.int32, (W, S), 0)
        oh_y = (ih == py).astype(jnp.float32)
        oh_x = (iw == px).astype(jnp.float32)
        gia = gia_ref[0]
        rows = lax.dot_general(gia, oh_y, (((0,), (0,)), ((), ())),
                               preferred_element_type=jnp.float32)
        g = jnp.sum(rows * oh_x, axis=0, keepdims=True)
        e = pr_ref[0] * g * in_view.astype(jnp.float32)
        flat = py * W + px
        ev_ref[0] = jnp.broadcast_to(e, (8, S))
        tv_ref[0] = jnp.broadcast_to(flat, (8, S))


_NW = 32
_NQ = 8
_QW = 512 * 512 // _NQ
_QR = 512 // _NQ


def _sc_body(ev_hbm, tv_hbm, out_hbm, acc0, acc1, evp, tvp, semi, sem0, sem1):
    cid = lax.axis_index("c")
    sid = lax.axis_index("s")
    wid = sid * 2 + cid
    zeros16 = jnp.zeros((16,), jnp.float32)
    bufs = (acc0, acc1)
    sems = (sem0, sem1)
    copies = [None, None]
    pref = []
    for p in range(4):
        bp = (wid + _NW * p) // _NQ
        pref.append(pltpu.async_copy(ev_hbm.at[bp], evp.at[p], semi))
        pref.append(pltpu.async_copy(tv_hbm.at[bp], tvp.at[p], semi))
    drained = False
    for p in range(4):
        acc = bufs[p % 2]
        if p >= 2:
            copies[p % 2].wait()
        item = wid + _NW * p
        b = item // _NQ
        q = item % _NQ

        def _zero(i, carry):
            for k in range(2):
                for m in range(32):
                    acc[i * 2 + k, pl.ds(m * 16, 16)] = zeros16
            return carry
        lax.fori_loop(0, _QR // 2, _zero, 0)

        if not drained:
            for cp in pref:
                cp.wait()
            drained = True
        qbase = q * _QW
        for i in range(16):
            csl = pl.ds(i * 16, 16)
            ei = evp[p, 0, csl]
            ti = tvp[p, 0, csl]
            local = ti - qbase
            mask = (ei != 0.0) & (local >= 0) & (local < _QW)
            lc = jnp.minimum(jnp.maximum(local, 0), _QW - 1)
            plsc.addupdate_scatter(acc, [lax.shift_right_logical(lc, 9),
                                         jnp.bitwise_and(lc, 511)],
                                   ei, mask=mask)
        copies[p % 2] = pltpu.async_copy(
            acc, out_hbm.at[b, pl.ds(q * _QR, _QR)], sems[p % 2])
    copies[0].wait()
    copies[1].wait()


def kernel(point_rates, spatial, gia, all_source_coords):
    B, H, W = gia.shape
    S = all_source_coords.shape[0]
    coords3 = jnp.transpose(all_source_coords)[None]
    pr3 = point_rates[:, None, :]
    ev, tv = pl.pallas_call(
        _tc_body,
        grid=(B,),
        in_specs=[
            pl.BlockSpec((1, 2, S), lambda b: (0, 0, 0)),
            pl.BlockSpec((1, 1, S), lambda b: (b, 0, 0)),
            pl.BlockSpec((1, 2, H, W), lambda b: (b, 0, 0, 0)),
            pl.BlockSpec((1, H, W), lambda b: (b, 0, 0)),
        ],
        out_specs=[
            pl.BlockSpec((1, 8, S), lambda b: (b, 0, 0)),
            pl.BlockSpec((1, 8, S), lambda b: (b, 0, 0)),
        ],
        out_shape=[
            jax.ShapeDtypeStruct((B, 8, S), jnp.float32),
            jax.ShapeDtypeStruct((B, 8, S), jnp.int32),
        ],
    )(coords3, pr3, spatial, gia)
    mesh = plsc.VectorSubcoreMesh(core_axis_name="c", subcore_axis_name="s")
    sc = pl.kernel(
        _sc_body,
        out_type=jax.ShapeDtypeStruct((B, H, W), jnp.float32),
        mesh=mesh,
        compiler_params=pltpu.CompilerParams(needs_layout_passes=False, skip_device_barrier=True),
        scratch_types=[
            pltpu.VMEM((_QR, 512), jnp.float32),
            pltpu.VMEM((_QR, 512), jnp.float32),
            pltpu.VMEM((4, 8, S), jnp.float32),
            pltpu.VMEM((4, 8, S), jnp.int32),
            pltpu.SemaphoreType.DMA,
            pltpu.SemaphoreType.DMA,
            pltpu.SemaphoreType.DMA,
        ],
    )
    out3 = sc(ev, tv)
    return out3[:, None]
